# Initial kernel scaffold; baseline (speedup 1.0000x reference)
import jax
import jax.numpy as jnp
from jax.experimental import pallas as pl


def kernel(xd, xd_ei, xd_batch, xt, xt_ei, xt_batch, y, params):
    raise NotImplementedError("write your pallas kernel here")


# speedup vs baseline: 3.2911x; 3.2911x over previous
import functools

import jax
import jax.numpy as jnp
from jax import lax
from jax.experimental import pallas as pl
from jax.experimental.pallas import tpu as pltpu
from jax.experimental.pallas import tpu_sc as plsc

_NSUB = 16


@functools.lru_cache(maxsize=None)
def _mk_edge_pass(Npad, Epad, C, G, _K):
    mesh = plsc.VectorSubcoreMesh(core_axis_name="c", subcore_axis_name="s")
    epw = Epad // _NSUB
    n_outer = epw // (_K * 128)
    rpw = Npad // _NSUB

    @functools.partial(
        pl.kernel,
        out_type=jax.ShapeDtypeStruct((G, Npad, C), jnp.float32),
        mesh=mesh,
        compiler_params=pltpu.CompilerParams(use_tc_tiling_on_sc=False),
        scratch_types=[
            pltpu.VMEM_SHARED((Npad, C), jnp.float32),
            pltpu.VMEM((_K, 128), jnp.int32),
            pltpu.VMEM((1, 128), jnp.int32),
            pltpu.VMEM((_K, 128), jnp.int32),
            pltpu.VMEM((128, C), jnp.float32),
        ],
    )
    def k(tbl, src2d, dst2d, zeros_h, out, acc, sidx, sidx2, didx, msg):
        c = lax.axis_index("c")
        s = lax.axis_index("s")
        r0 = s * rpw
        erow0 = s * (epw // 128)
        for p in range((G + 1) // 2):
            g = 2 * p + c

            @pl.when(g < G)
            def _pass():
                pltpu.sync_copy(zeros_h, acc.at[pl.ds(r0, rpw), :])
                plsc.subcore_barrier()

                @pl.loop(0, n_outer)
                def _chunk(i):
                    row = erow0 + i * _K
                    pltpu.sync_copy(src2d.at[pl.ds(row, _K), :], sidx)
                    pltpu.sync_copy(dst2d.at[pl.ds(row, _K), :], didx)
                    for j in range(_K):
                        for l in range(8):
                            sidx2[0, pl.ds(l * 16, 16)] = (
                                sidx[j, pl.ds(l * 16, 16)] + g * Npad
                            )
                        pltpu.sync_copy(tbl.at[sidx2.at[0]], msg)
                        pltpu.sync_copy(msg, acc.at[didx.at[j]], add=True)

                plsc.subcore_barrier()
                pltpu.sync_copy(
                    acc.at[pl.ds(r0, rpw), :], out.at[g, pl.ds(r0, rpw), :]
                )

    return k


@functools.lru_cache(maxsize=None)
def _mk_deg(Npad, Epad):
    mesh = plsc.VectorSubcoreMesh(core_axis_name="c", subcore_axis_name="s")
    KD = 8
    epw = Epad // 32
    n_outer = epw // (KD * 128)
    rpw = Npad // _NSUB

    @functools.partial(
        pl.kernel,
        out_type=jax.ShapeDtypeStruct((2, Npad, 16), jnp.float32),
        mesh=mesh,
        compiler_params=pltpu.CompilerParams(use_tc_tiling_on_sc=False),
        scratch_types=[
            pltpu.VMEM_SHARED((Npad, 16), jnp.float32),
            pltpu.VMEM((KD, 128), jnp.int32),
            pltpu.VMEM((128, 16), jnp.float32),
        ],
    )
    def k(dst2d, zeros_h, ones_h, out, acc, didx, msg):
        c = lax.axis_index("c")
        s = lax.axis_index("s")
        wid = c * _NSUB + s
        r0 = s * rpw
        erow0 = wid * (epw // 128)
        pltpu.sync_copy(ones_h, msg)
        pltpu.sync_copy(zeros_h, acc.at[pl.ds(r0, rpw), :])
        plsc.subcore_barrier()

        @pl.loop(0, n_outer)
        def _chunk(i):
            row = erow0 + i * KD
            pltpu.sync_copy(dst2d.at[pl.ds(row, KD), :], didx)
            for j in range(KD):
                pltpu.sync_copy(msg, acc.at[didx.at[j]], add=True)

        plsc.subcore_barrier()
        pltpu.sync_copy(
            acc.at[pl.ds(r0, rpw), :], out.at[c, pl.ds(r0, rpw), :]
        )

    return k


_BR = 512


def _matmul_scale(x, wtp, da, db):
    Gin, Npad, Cin = x.shape
    Gout, Kp, C = wtp.shape

    def body(x_ref, w_ref, da_ref, db_ref, o_ref):
        dinv = lax.rsqrt(da_ref[:, :1] + db_ref[:, :1] + 1.0)
        w = w_ref[0]
        acc = jnp.dot(
            x_ref[0], w[:Cin, :], preferred_element_type=jnp.float32
        )
        for gi in range(1, Gin):
            acc += jnp.dot(
                x_ref[gi], w[gi * Cin : (gi + 1) * Cin, :],
                preferred_element_type=jnp.float32,
            )
        o_ref[...] = (acc * dinv)[None]

    return pl.pallas_call(
        body,
        grid=(Npad // _BR, Gout),
        in_specs=[
            pl.BlockSpec((Gin, _BR, Cin), lambda r, g: (0, r, 0)),
            pl.BlockSpec((1, Kp, C), lambda r, g: (g, 0, 0)),
            pl.BlockSpec((_BR, 16), lambda r, g: (r, 0)),
            pl.BlockSpec((_BR, 16), lambda r, g: (r, 0)),
        ],
        out_specs=pl.BlockSpec((1, _BR, C), lambda r, g: (g, r, 0)),
        out_shape=jax.ShapeDtypeStruct((Gout, Npad, C), jnp.float32),
    )(x, wtp, da, db)


def _finish(agg, hs, da, db, bias):
    G, Npad, C = agg.shape

    def body(a_ref, h_ref, da_ref, db_ref, b_ref, o_ref):
        dinv = lax.rsqrt(da_ref[:, :1] + db_ref[:, :1] + 1.0)
        o_ref[...] = jnp.maximum(
            dinv * (a_ref[0] + h_ref[0]) + b_ref[0], 0.0
        )[None]

    return pl.pallas_call(
        body,
        grid=(Npad // _BR, G),
        in_specs=[
            pl.BlockSpec((1, _BR, C), lambda r, g: (g, r, 0)),
            pl.BlockSpec((1, _BR, C), lambda r, g: (g, r, 0)),
            pl.BlockSpec((_BR, 16), lambda r, g: (r, 0)),
            pl.BlockSpec((_BR, 16), lambda r, g: (r, 0)),
            pl.BlockSpec((1, 1, C), lambda r, g: (g, 0, 0)),
        ],
        out_specs=pl.BlockSpec((1, _BR, C), lambda r, g: (g, r, 0)),
        out_shape=jax.ShapeDtypeStruct((G, Npad, C), jnp.float32),
    )(agg, hs, da, db, bias)


def _pool(h, batch, B):
    G, Npad, C = h.shape

    def body(h_ref, b_ref, s_ref, c_ref):
        r = pl.program_id(0)
        ids = b_ref[...]
        iota = lax.broadcasted_iota(jnp.int32, (_BR, B), 1)
        oh = (ids == iota).astype(jnp.float32)
        st = jnp.stack(
            [
                lax.dot_general(
                    h_ref[g], oh, (((0,), (0,)), ((), ())),
                    preferred_element_type=jnp.float32,
                )
                for g in range(G)
            ]
        )
        cs = jnp.broadcast_to(jnp.sum(oh, axis=0, keepdims=True), (8, B))

        @pl.when(r == 0)
        def _():
            s_ref[...] = st
            c_ref[...] = cs

        @pl.when(r > 0)
        def _():
            s_ref[...] += st
            c_ref[...] += cs

    return pl.pallas_call(
        body,
        grid=(Npad // _BR,),
        in_specs=[
            pl.BlockSpec((G, _BR, C), lambda r: (0, r, 0)),
            pl.BlockSpec((_BR, 1), lambda r: (r, 0)),
        ],
        out_specs=[
            pl.BlockSpec((G, C, B), lambda r: (0, 0, 0)),
            pl.BlockSpec((8, B), lambda r: (0, 0)),
        ],
        out_shape=[
            jax.ShapeDtypeStruct((G, C, B), jnp.float32),
            jax.ShapeDtypeStruct((8, B), jnp.float32),
        ],
    )(h, batch)


def _head(sd, cd, sp, cp, ws):
    B = sd.shape[1]

    def body(sd_ref, cd_ref, sp_ref, cp_ref,
             mg1w, mg1b, mg2w, mg2b, pg1w, pg1b, pg2w, pg2b,
             c1w, c1b, c2w, c2b, c3w, c3b, o_ref):
        def dot(a, b):
            return jnp.dot(a, b, preferred_element_type=jnp.float32)

        md = sd_ref[...] / jnp.maximum(cd_ref[:1], 1.0)
        gd = jnp.maximum(dot(mg1w[...], md) + mg1b[...], 0.0)
        gd = dot(mg2w[...], gd) + mg2b[...]
        mp = sp_ref[...] / jnp.maximum(cp_ref[:1], 1.0)
        gt = jnp.maximum(dot(pg1w[...], mp) + pg1b[...], 0.0)
        gt = dot(pg2w[...], gt) + pg2b[...]
        xj = jnp.concatenate([gd, gt], axis=0)
        c1 = jnp.maximum(dot(c1w[...], xj) + c1b[...], 0.0)
        c2 = jnp.maximum(dot(c2w[...], c1) + c2b[...], 0.0)
        o_ref[...] = dot(c3w[...], c2) + c3b[...]

    return pl.pallas_call(
        body,
        out_shape=jax.ShapeDtypeStruct((8, B), jnp.float32),
    )(sd, cd, sp, cp, *ws)


def _pad2(a, r, c):
    out = jnp.zeros((r, c), jnp.float32)
    return out.at[: a.shape[0], : a.shape[1]].set(a)


def _branch(x, ei, batch, Ws, bs, Npad, Epad, C, Gs, B):
    N = x.shape[0]
    E = ei.shape[1]
    pad_idx = jnp.full((Epad - E,), Npad - 1, jnp.int32)
    src2d = jnp.concatenate([ei[0], pad_idx]).reshape(-1, 128)
    dst2d = jnp.concatenate([ei[1], pad_idx]).reshape(-1, 128)
    batch_p = jnp.concatenate(
        [batch, jnp.full((Npad - N,), B, jnp.int32)]
    ).reshape(Npad, 1)

    zeros_h = jnp.zeros((Npad // _NSUB, C), jnp.float32)
    deg = _mk_deg(Npad, Epad)(
        dst2d,
        jnp.zeros((Npad // _NSUB, 16), jnp.float32),
        jnp.ones((128, 16), jnp.float32),
    )
    da, db = deg[0], deg[1]

    Kp = ((x.shape[1] + 7) // 8) * 8
    h = _pad2(x, Npad, Kp).reshape(1, Npad, Kp)
    for li in range(3):
        G = Gs[li]
        Dp = G * C
        Kin = h.shape[0] * h.shape[2]
        wtp = _pad2(Ws[li].T, Kin, Dp).reshape(Kin, G, C).transpose(1, 0, 2)
        bias = _pad2(bs[li].reshape(1, -1), 1, Dp).reshape(G, 1, C)
        hs = _matmul_scale(h, wtp, da, db)
        agg = _mk_edge_pass(Npad, Epad, C, G, 128 // C)(
            hs.reshape(Npad * G, C), src2d, dst2d, zeros_h
        )
        h = _finish(agg, hs, da, db, bias)

    return _pool(h, batch_p, B)


def _fwd(xd, xd_ei, xd_batch, xt, xt_ei, xt_batch, y, params):
    p = params
    B = y.shape[0]
    sd, cd = _branch(
        xd, xd_ei, xd_batch,
        (p["mW1"], p["mW2"], p["mW3"]), (p["mb1"], p["mb2"], p["mb3"]),
        51200, 819200, 32, (2, 4, 7), B,
    )
    sp, cp = _branch(
        xt, xt_ei, xt_batch,
        (p["pW1"], p["pW2"], p["pW3"]), (p["pb1"], p["pb2"], p["pb3"]),
        100352, 1638400, 16, (3, 6, 11), B,
    )

    def col(v):
        return v.reshape(-1, 1)

    ws = [
        _pad2(p["mg1W"], 1024, 224), col(p["mg1b"]),
        p["mg2W"], col(p["mg2b"]),
        _pad2(p["pg1W"], 1024, 176), col(p["pg1b"]),
        p["pg2W"], col(p["pg2b"]),
        p["c1W"], col(p["c1b"]),
        p["c2W"], col(p["c2b"]),
        jnp.broadcast_to(p["c3W"], (8, 512)), jnp.full((8, 1), p["c3b"][0]),
    ]
    out = _head(sd.reshape(-1, B), cd, sp.reshape(-1, B), cp, ws)
    return out[0]


_fwd_jit = jax.jit(_fwd)


def kernel(xd, xd_ei, xd_batch, xt, xt_ei, xt_batch, y, params):
    return (_fwd_jit(xd, xd_ei, xd_batch, xt, xt_ei, xt_batch, y, params), y)


# speedup vs baseline: 3.6677x; 1.1144x over previous
import functools

import jax
import jax.numpy as jnp
from jax import lax
from jax.experimental import pallas as pl
from jax.experimental.pallas import tpu as pltpu
from jax.experimental.pallas import tpu_sc as plsc

_NSUB = 16


@functools.lru_cache(maxsize=None)
def _mk_edge_pass(Npad, Epad, C, G, _K):
    mesh = plsc.VectorSubcoreMesh(core_axis_name="c", subcore_axis_name="s")
    epw = Epad // _NSUB
    n_outer = epw // (_K * 64)
    rpw = Npad // _NSUB

    @functools.partial(
        pl.kernel,
        out_type=jax.ShapeDtypeStruct((G, Npad, C), jnp.float32),
        mesh=mesh,
        compiler_params=pltpu.CompilerParams(use_tc_tiling_on_sc=False),
        scratch_types=[
            pltpu.VMEM_SHARED((Npad, C), jnp.float32),
            pltpu.VMEM((_K, 64), jnp.int32),
            pltpu.VMEM((_K, 64), jnp.int32),
            pltpu.VMEM((64, C), jnp.float32),
            pltpu.VMEM((64, C), jnp.float32),
            pltpu.SemaphoreType.DMA,
            pltpu.SemaphoreType.DMA,
        ],
    )
    def k(tbl, src2d, dst2d, zeros_h, out, acc, sidx, didx, msga, msgb, sem0, sem1):
        c = lax.axis_index("c")
        s = lax.axis_index("s")
        r0 = s * rpw
        erow0 = s * (epw // 64)
        for p in range((G + 1) // 2):
            g = 2 * p + c

            @pl.when(g < G)
            def _pass():
                pltpu.sync_copy(zeros_h, acc.at[pl.ds(r0, rpw), :])
                plsc.subcore_barrier()

                @pl.loop(0, n_outer)
                def _chunk(i):
                    row = erow0 + i * _K
                    pltpu.sync_copy(src2d.at[pl.ds(row, _K), :], sidx)
                    pltpu.sync_copy(dst2d.at[pl.ds(row, _K), :], didx)
                    for j in range(_K):
                        for l in range(4):
                            sidx[j, pl.ds(l * 16, 16)] = (
                                sidx[j, pl.ds(l * 16, 16)] + g * Npad
                            )
                    bufs = (msga, msgb)
                    sems = (sem0, sem1)
                    pend = pltpu.async_copy(tbl.at[sidx.at[0]], bufs[0], sems[0])
                    for j in range(_K):
                        nxt = None
                        if j + 1 < _K:
                            nxt = pltpu.async_copy(
                                tbl.at[sidx.at[j + 1]],
                                bufs[(j + 1) % 2],
                                sems[(j + 1) % 2],
                            )
                        pend.wait()
                        pltpu.sync_copy(
                            bufs[j % 2], acc.at[didx.at[j]], add=True
                        )
                        pend = nxt

                plsc.subcore_barrier()
                pltpu.sync_copy(
                    acc.at[pl.ds(r0, rpw), :], out.at[g, pl.ds(r0, rpw), :]
                )

    return k


@functools.lru_cache(maxsize=None)
def _mk_deg(Npad, Epad):
    mesh = plsc.VectorSubcoreMesh(core_axis_name="c", subcore_axis_name="s")
    KD = 8
    epw = Epad // 32
    n_outer = epw // (KD * 128)
    rpw = Npad // _NSUB

    @functools.partial(
        pl.kernel,
        out_type=jax.ShapeDtypeStruct((2, Npad, 16), jnp.float32),
        mesh=mesh,
        compiler_params=pltpu.CompilerParams(use_tc_tiling_on_sc=False),
        scratch_types=[
            pltpu.VMEM_SHARED((Npad, 16), jnp.float32),
            pltpu.VMEM((KD, 128), jnp.int32),
            pltpu.VMEM((128, 16), jnp.float32),
        ],
    )
    def k(dst2d, zeros_h, ones_h, out, acc, didx, msg):
        c = lax.axis_index("c")
        s = lax.axis_index("s")
        wid = c * _NSUB + s
        r0 = s * rpw
        erow0 = wid * (epw // 128)
        pltpu.sync_copy(ones_h, msg)
        pltpu.sync_copy(zeros_h, acc.at[pl.ds(r0, rpw), :])
        plsc.subcore_barrier()

        @pl.loop(0, n_outer)
        def _chunk(i):
            row = erow0 + i * KD
            pltpu.sync_copy(dst2d.at[pl.ds(row, KD), :], didx)
            for j in range(KD):
                pltpu.sync_copy(msg, acc.at[didx.at[j]], add=True)

        plsc.subcore_barrier()
        pltpu.sync_copy(
            acc.at[pl.ds(r0, rpw), :], out.at[c, pl.ds(r0, rpw), :]
        )

    return k


_BR = 512


def _matmul_scale(x, wtp, da, db):
    Gin, Npad, Cin = x.shape
    Gout, Kp, C = wtp.shape

    def body(x_ref, w_ref, da_ref, db_ref, o_ref):
        dinv = lax.rsqrt(da_ref[:, :1] + db_ref[:, :1] + 1.0)
        w = w_ref[0]
        acc = jnp.dot(
            x_ref[0], w[:Cin, :], preferred_element_type=jnp.float32
        )
        for gi in range(1, Gin):
            acc += jnp.dot(
                x_ref[gi], w[gi * Cin : (gi + 1) * Cin, :],
                preferred_element_type=jnp.float32,
            )
        o_ref[...] = (acc * dinv)[None]

    return pl.pallas_call(
        body,
        grid=(Npad // _BR, Gout),
        in_specs=[
            pl.BlockSpec((Gin, _BR, Cin), lambda r, g: (0, r, 0)),
            pl.BlockSpec((1, Kp, C), lambda r, g: (g, 0, 0)),
            pl.BlockSpec((_BR, 16), lambda r, g: (r, 0)),
            pl.BlockSpec((_BR, 16), lambda r, g: (r, 0)),
        ],
        out_specs=pl.BlockSpec((1, _BR, C), lambda r, g: (g, r, 0)),
        out_shape=jax.ShapeDtypeStruct((Gout, Npad, C), jnp.float32),
    )(x, wtp, da, db)


def _finish(agg, hs, da, db, bias):
    G, Npad, C = agg.shape

    def body(a_ref, h_ref, da_ref, db_ref, b_ref, o_ref):
        dinv = lax.rsqrt(da_ref[:, :1] + db_ref[:, :1] + 1.0)
        o_ref[...] = jnp.maximum(
            dinv * (a_ref[0] + h_ref[0]) + b_ref[0], 0.0
        )[None]

    return pl.pallas_call(
        body,
        grid=(Npad // _BR, G),
        in_specs=[
            pl.BlockSpec((1, _BR, C), lambda r, g: (g, r, 0)),
            pl.BlockSpec((1, _BR, C), lambda r, g: (g, r, 0)),
            pl.BlockSpec((_BR, 16), lambda r, g: (r, 0)),
            pl.BlockSpec((_BR, 16), lambda r, g: (r, 0)),
            pl.BlockSpec((1, 1, C), lambda r, g: (g, 0, 0)),
        ],
        out_specs=pl.BlockSpec((1, _BR, C), lambda r, g: (g, r, 0)),
        out_shape=jax.ShapeDtypeStruct((G, Npad, C), jnp.float32),
    )(agg, hs, da, db, bias)


def _pool(h, batch, B):
    G, Npad, C = h.shape

    def body(h_ref, b_ref, s_ref, c_ref):
        r = pl.program_id(0)
        ids = b_ref[...]
        iota = lax.broadcasted_iota(jnp.int32, (_BR, B), 1)
        oh = (ids == iota).astype(jnp.float32)
        st = jnp.stack(
            [
                lax.dot_general(
                    h_ref[g], oh, (((0,), (0,)), ((), ())),
                    preferred_element_type=jnp.float32,
                )
                for g in range(G)
            ]
        )
        cs = jnp.broadcast_to(jnp.sum(oh, axis=0, keepdims=True), (8, B))

        @pl.when(r == 0)
        def _():
            s_ref[...] = st
            c_ref[...] = cs

        @pl.when(r > 0)
        def _():
            s_ref[...] += st
            c_ref[...] += cs

    return pl.pallas_call(
        body,
        grid=(Npad // _BR,),
        in_specs=[
            pl.BlockSpec((G, _BR, C), lambda r: (0, r, 0)),
            pl.BlockSpec((_BR, 1), lambda r: (r, 0)),
        ],
        out_specs=[
            pl.BlockSpec((G, C, B), lambda r: (0, 0, 0)),
            pl.BlockSpec((8, B), lambda r: (0, 0)),
        ],
        out_shape=[
            jax.ShapeDtypeStruct((G, C, B), jnp.float32),
            jax.ShapeDtypeStruct((8, B), jnp.float32),
        ],
    )(h, batch)


def _head(sd, cd, sp, cp, ws):
    B = sd.shape[1]

    def body(sd_ref, cd_ref, sp_ref, cp_ref,
             mg1w, mg1b, mg2w, mg2b, pg1w, pg1b, pg2w, pg2b,
             c1w, c1b, c2w, c2b, c3w, c3b, o_ref):
        def dot(a, b):
            return jnp.dot(a, b, preferred_element_type=jnp.float32)

        md = sd_ref[...] / jnp.maximum(cd_ref[:1], 1.0)
        gd = jnp.maximum(dot(mg1w[...], md) + mg1b[...], 0.0)
        gd = dot(mg2w[...], gd) + mg2b[...]
        mp = sp_ref[...] / jnp.maximum(cp_ref[:1], 1.0)
        gt = jnp.maximum(dot(pg1w[...], mp) + pg1b[...], 0.0)
        gt = dot(pg2w[...], gt) + pg2b[...]
        xj = jnp.concatenate([gd, gt], axis=0)
        c1 = jnp.maximum(dot(c1w[...], xj) + c1b[...], 0.0)
        c2 = jnp.maximum(dot(c2w[...], c1) + c2b[...], 0.0)
        o_ref[...] = dot(c3w[...], c2) + c3b[...]

    return pl.pallas_call(
        body,
        out_shape=jax.ShapeDtypeStruct((8, B), jnp.float32),
    )(sd, cd, sp, cp, *ws)


def _pad2(a, r, c):
    out = jnp.zeros((r, c), jnp.float32)
    return out.at[: a.shape[0], : a.shape[1]].set(a)


def _branch(x, ei, batch, Ws, bs, Npad, Epad, C, Gs, B):
    N = x.shape[0]
    E = ei.shape[1]
    pad_idx = jnp.full((Epad - E,), Npad - 1, jnp.int32)
    src_p = jnp.concatenate([ei[0], pad_idx])
    dst_p = jnp.concatenate([ei[1], pad_idx])
    src2d = src_p.reshape(-1, 64)
    dst2d = dst_p.reshape(-1, 64)
    dst2d_w = dst_p.reshape(-1, 128)
    batch_p = jnp.concatenate(
        [batch, jnp.full((Npad - N,), B, jnp.int32)]
    ).reshape(Npad, 1)

    zeros_h = jnp.zeros((Npad // _NSUB, C), jnp.float32)
    deg = _mk_deg(Npad, Epad)(
        dst2d_w,
        jnp.zeros((Npad // _NSUB, 16), jnp.float32),
        jnp.ones((128, 16), jnp.float32),
    )
    da, db = deg[0], deg[1]

    Kp = ((x.shape[1] + 7) // 8) * 8
    h = _pad2(x, Npad, Kp).reshape(1, Npad, Kp)
    for li in range(3):
        G = Gs[li]
        Dp = G * C
        Kin = h.shape[0] * h.shape[2]
        wtp = _pad2(Ws[li].T, Kin, Dp).reshape(Kin, G, C).transpose(1, 0, 2)
        bias = _pad2(bs[li].reshape(1, -1), 1, Dp).reshape(G, 1, C)
        hs = _matmul_scale(h, wtp, da, db)
        agg = _mk_edge_pass(Npad, Epad, C, G, 16)(
            hs.reshape(Npad * G, C), src2d, dst2d, zeros_h
        )
        h = _finish(agg, hs, da, db, bias)

    return _pool(h, batch_p, B)


def _fwd(xd, xd_ei, xd_batch, xt, xt_ei, xt_batch, y, params):
    p = params
    B = y.shape[0]
    sd, cd = _branch(
        xd, xd_ei, xd_batch,
        (p["mW1"], p["mW2"], p["mW3"]), (p["mb1"], p["mb2"], p["mb3"]),
        51200, 819200, 32, (2, 4, 7), B,
    )
    sp, cp = _branch(
        xt, xt_ei, xt_batch,
        (p["pW1"], p["pW2"], p["pW3"]), (p["pb1"], p["pb2"], p["pb3"]),
        100352, 1638400, 16, (3, 6, 11), B,
    )

    def col(v):
        return v.reshape(-1, 1)

    ws = [
        _pad2(p["mg1W"], 1024, 224), col(p["mg1b"]),
        p["mg2W"], col(p["mg2b"]),
        _pad2(p["pg1W"], 1024, 176), col(p["pg1b"]),
        p["pg2W"], col(p["pg2b"]),
        p["c1W"], col(p["c1b"]),
        p["c2W"], col(p["c2b"]),
        jnp.broadcast_to(p["c3W"], (8, 512)), jnp.full((8, 1), p["c3b"][0]),
    ]
    out = _head(sd.reshape(-1, B), cd, sp.reshape(-1, B), cp, ws)
    return out[0]


_fwd_jit = jax.jit(_fwd)


def kernel(xd, xd_ei, xd_batch, xt, xt_ei, xt_batch, y, params):
    return (_fwd_jit(xd, xd_ei, xd_batch, xt, xt_ei, xt_batch, y, params), y)
